# Initial kernel scaffold; baseline (speedup 1.0000x reference)
import jax
import jax.numpy as jnp
from jax.experimental import pallas as pl


def kernel(x, W_emb1, b_emb1, W_emb2, b_emb2, W_msg, W_upd):
    raise NotImplementedError("write your pallas kernel here")


# speedup vs baseline: 9.2482x; 9.2482x over previous
import functools

import jax
import jax.numpy as jnp
from jax import lax
from jax.experimental import pallas as pl

_F32_MAX_BITS = 0x7F7FFFFF


def _mm(a, b):
    return lax.dot_general(
        a, b, (((1,), (0,)), ((), ())),
        precision=lax.Precision.HIGHEST,
        preferred_element_type=jnp.float32,
    )


def _gram(a, b):
    return lax.dot_general(
        a, b, (((1,), (1,)), ((), ())),
        precision=lax.Precision.HIGHEST,
        preferred_element_type=jnp.float32,
    )


def _topk_mask(S, k):
    n = S.shape[0]
    lo = jnp.zeros((n, 1), jnp.int32)
    hi = jnp.full((n, 1), _F32_MAX_BITS, jnp.int32)

    def body(_, carry):
        lo, hi = carry
        mid = lo + (hi - lo + 1) // 2
        t = lax.bitcast_convert_type(mid, jnp.float32)
        cnt = jnp.sum((S >= t).astype(jnp.float32), axis=1, keepdims=True)
        ge = cnt >= float(k)
        return jnp.where(ge, mid, lo), jnp.where(ge, hi, mid - 1)

    lo, _ = lax.fori_loop(0, 31, body, (lo, hi))
    t = lax.bitcast_convert_type(lo, jnp.float32)
    return jnp.where(S >= t, S, 0.0)


def _embed_body(x_ref, w1_ref, b1_ref, w2_ref, b2_ref, h_ref):
    h = jax.nn.relu(_mm(x_ref[0], w1_ref[...]) + b1_ref[...])
    h_ref[0] = jax.nn.relu(_mm(h, w2_ref[...]) + b2_ref[...])


def _msgpass_body(hrows_ref, hfull_ref, wu_ref, wm_ref, out_ref, *, k):
    hr = hrows_ref[0]
    hb = hfull_ref[0]
    A = _topk_mask(_gram(hr, hb), k)
    m = _mm(A, hb)
    out_ref[0] = jax.nn.relu(_mm(hr, wu_ref[...]) + _mm(m, wm_ref[...]))


def _final_body(hrows_ref, hfull_ref, out_ref, *, k):
    out_ref[0] = _topk_mask(_gram(hrows_ref[0], hfull_ref[0]), k)


def kernel(x, W_emb1, b_emb1, W_emb2, b_emb2, W_msg, W_upd):
    bs, n, f = x.shape
    hid = W_emb1.shape[1]
    iters = W_msg.shape[0]
    k = 50
    r = min(512, n)
    rb = n // r

    b1 = b_emb1.reshape(1, hid)
    b2 = b_emb2.reshape(1, hid)

    h = pl.pallas_call(
        _embed_body,
        grid=(bs,),
        in_specs=[
            pl.BlockSpec((1, n, f), lambda b: (b, 0, 0)),
            pl.BlockSpec((f, hid), lambda b: (0, 0)),
            pl.BlockSpec((1, hid), lambda b: (0, 0)),
            pl.BlockSpec((hid, hid), lambda b: (0, 0)),
            pl.BlockSpec((1, hid), lambda b: (0, 0)),
        ],
        out_specs=pl.BlockSpec((1, n, hid), lambda b: (b, 0, 0)),
        out_shape=jax.ShapeDtypeStruct((bs, n, hid), jnp.float32),
    )(x, W_emb1, b1, W_emb2, b2)

    for i in range(iters):
        h = pl.pallas_call(
            functools.partial(_msgpass_body, k=k),
            grid=(bs, rb),
            in_specs=[
                pl.BlockSpec((1, r, hid), lambda b, j: (b, j, 0)),
                pl.BlockSpec((1, n, hid), lambda b, j: (b, 0, 0)),
                pl.BlockSpec((hid, hid), lambda b, j: (0, 0)),
                pl.BlockSpec((hid, hid), lambda b, j: (0, 0)),
            ],
            out_specs=pl.BlockSpec((1, r, hid), lambda b, j: (b, j, 0)),
            out_shape=jax.ShapeDtypeStruct((bs, n, hid), jnp.float32),
        )(h, h, W_upd[i], W_msg[i])

    return pl.pallas_call(
        functools.partial(_final_body, k=k),
        grid=(bs, rb),
        in_specs=[
            pl.BlockSpec((1, r, hid), lambda b, j: (b, j, 0)),
            pl.BlockSpec((1, n, hid), lambda b, j: (b, 0, 0)),
        ],
        out_specs=pl.BlockSpec((1, r, n), lambda b, j: (b, j, 0)),
        out_shape=jax.ShapeDtypeStruct((bs, n, n), jnp.float32),
    )(h, h)
